# baseline (speedup 1.0000x reference)
import functools

import jax
import jax.numpy as jnp
from jax import lax
from jax.experimental import pallas as pl
from jax.experimental.pallas import tpu as pltpu
from jax.experimental.pallas import tpu_sc as plsc

_PREC = lax.Precision.HIGHEST
_NW = 32
_CHUNK = 64
_GRAN = _NW * _CHUNK * 2


def _silu(x):
    return x / (1.0 + jnp.exp(-x))


def _row_block(n, cap):
    b = 8
    for c in range(8, cap + 1, 8):
        if n % c == 0:
            b = c
    return b


def _rowmm(x, w, b, xpad, splits):
    n, d = x.shape
    m = w.shape[1]
    dx = xpad.shape[1]
    bn = _row_block(n, 2048)
    sw = m // splits

    def body(x_ref, w_ref, b_ref, xp_ref, *outs):
        acc = jnp.dot(x_ref[...], w_ref[...], precision=_PREC,
                      preferred_element_type=jnp.float32) + b_ref[...]
        xp = xp_ref[...]
        for j, o in enumerate(outs):
            o[...] = jnp.concatenate([acc[:, j * sw:(j + 1) * sw], xp], axis=1)

    outs = tuple(jax.ShapeDtypeStruct((n, sw + dx), jnp.float32)
                 for _ in range(splits))
    return pl.pallas_call(
        body,
        grid=(n // bn,),
        in_specs=[
            pl.BlockSpec((bn, d), lambda i: (i, 0)),
            pl.BlockSpec((d, m), lambda i: (0, 0)),
            pl.BlockSpec((1, m), lambda i: (0, 0)),
            pl.BlockSpec((bn, dx), lambda i: (i, 0)),
        ],
        out_specs=tuple(pl.BlockSpec((bn, sw + dx), lambda i: (i, 0))
                        for _ in range(splits)),
        out_shape=outs,
    )(x, w, b.reshape(1, m), xpad)


def _sc_gather(t_src_ll, t_dst_ll, src_ll, dst_ll,
               t_src_rl, t_dst_rl, src_rl, dst_rl):
    e_ll = src_ll.shape[0]
    e_rl = src_rl.shape[0]
    w = t_src_ll.shape[1]
    mesh = plsc.VectorSubcoreMesh(core_axis_name="c", subcore_axis_name="s")

    @functools.partial(
        pl.kernel,
        out_type=(jax.ShapeDtypeStruct((e_ll, w), jnp.float32),
                  jax.ShapeDtypeStruct((e_ll, w), jnp.float32),
                  jax.ShapeDtypeStruct((e_rl, w), jnp.float32),
                  jax.ShapeDtypeStruct((e_rl, w), jnp.float32)),
        mesh=mesh,
        scratch_types=[
            pltpu.VMEM((2, _CHUNK), jnp.int32),
            pltpu.VMEM((2, _CHUNK), jnp.int32),
            pltpu.VMEM((_CHUNK, w), jnp.float32),
            pltpu.VMEM((_CHUNK, w), jnp.float32),
            pltpu.VMEM((_CHUNK, w), jnp.float32),
            pltpu.VMEM((_CHUNK, w), jnp.float32),
            pltpu.SemaphoreType.DMA,
            pltpu.SemaphoreType.DMA,
            pltpu.SemaphoreType.DMA,
            pltpu.SemaphoreType.DMA,
        ],
    )
    def k(tsll_h, tdll_h, sll_h, dll_h, tsrl_h, tdrl_h, srl_h, drl_h,
          gsll_h, gdll_h, gsrl_h, gdrl_h,
          idx_s, idx_d, rs0, rd0, rs1, rd1, sg0, sg1, sw0, sw1):
        wid = lax.axis_index("c") * 16 + lax.axis_index("s")
        slots = ((rs0, rd0, sg0, sw0), (rs1, rd1, sg1, sw1))

        def run(tsrc_h, tdst_h, src_h, dst_h, gs_h, gd_h, e):
            e_per_w = e // _NW
            n_chunks = e_per_w // _CHUNK
            base_w = wid * e_per_w

            def issue(slot, i, reclaim):
                rs, rd, sg, sw = slots[slot]
                base = base_w + i * _CHUNK
                if reclaim:
                    pltpu.make_async_copy(rs, gs_h.at[pl.ds(base, _CHUNK)], sw).wait()
                    pltpu.make_async_copy(rd, gd_h.at[pl.ds(base, _CHUNK)], sw).wait()
                pltpu.sync_copy(src_h.at[pl.ds(base, _CHUNK)], idx_s.at[slot])
                pltpu.sync_copy(dst_h.at[pl.ds(base, _CHUNK)], idx_d.at[slot])
                pltpu.async_copy(tsrc_h.at[idx_s.at[slot]], rs, sg)
                pltpu.async_copy(tdst_h.at[idx_d.at[slot]], rd, sg)

            def complete(slot, i):
                rs, rd, sg, sw = slots[slot]
                base = base_w + i * _CHUNK
                pltpu.make_async_copy(tsrc_h.at[idx_s.at[slot]], rs, sg).wait()
                pltpu.make_async_copy(tdst_h.at[idx_d.at[slot]], rd, sg).wait()
                pltpu.async_copy(rs, gs_h.at[pl.ds(base, _CHUNK)], sw)
                pltpu.async_copy(rd, gd_h.at[pl.ds(base, _CHUNK)], sw)

            issue(0, 0, False)
            issue(1, 1, False)

            def pair(p, carry):
                complete(0, 2 * p)
                issue(0, 2 * p + 2, True)
                complete(1, 2 * p + 1)
                issue(1, 2 * p + 3, True)
                return carry

            lax.fori_loop(0, n_chunks // 2 - 1, pair, 0)
            complete(0, n_chunks - 2)
            complete(1, n_chunks - 1)
            for rs, rd, sg, sw in slots:
                pltpu.make_async_copy(rs, gs_h.at[pl.ds(base_w, _CHUNK)], sw).wait()
                pltpu.make_async_copy(rd, gd_h.at[pl.ds(base_w, _CHUNK)], sw).wait()

        run(tsll_h, tdll_h, sll_h, dll_h, gsll_h, gdll_h, e_ll)
        run(tsrl_h, tdrl_h, srl_h, drl_h, gsrl_h, gdrl_h, e_rl)

    return k(t_src_ll, t_dst_ll, src_ll, dst_ll,
             t_src_rl, t_dst_rl, src_rl, dst_rl)


def _tc_edge(gs, gd, w2e, b2e, w2c, b2c, wde, wdc):
    e, w = gs.shape
    h = 128
    be = 1024

    def body(gs_ref, gd_ref, w2e_ref, b2e_ref, w2c_ref,
             b2c_ref, wde_ref, wdc_ref, oh_ref, ox_ref):
        gsv = gs_ref[...]
        gdv = gd_ref[...]
        xdiff = gsv[:, 2 * h:] - gdv[:, 2 * h:]
        d2 = jnp.sum(xdiff * xdiff, axis=1, keepdims=True)
        dij = jnp.sqrt(d2)
        xn = xdiff / (dij + 1e-9)
        ue = gsv[:, :h] + gdv[:, :h] + dij * wde_ref[...]
        uc = gsv[:, h:2 * h] + gdv[:, h:2 * h] + dij * wdc_ref[...]
        a = _silu(ue)
        mh = _silu(jnp.dot(a, w2e_ref[...], precision=_PREC,
                           preferred_element_type=jnp.float32) + b2e_ref[...])
        c = _silu(uc)
        s = _silu(jnp.sum(c * w2c_ref[...], axis=1, keepdims=True) + b2c_ref[...])
        oh_ref[...] = mh
        ox_ref[...] = s * xn

    return pl.pallas_call(
        body,
        grid=(e // be,),
        in_specs=[
            pl.BlockSpec((be, w), lambda i: (i, 0)),
            pl.BlockSpec((be, w), lambda i: (i, 0)),
            pl.BlockSpec((h, h), lambda i: (0, 0)),
            pl.BlockSpec((1, h), lambda i: (0, 0)),
            pl.BlockSpec((1, h), lambda i: (0, 0)),
            pl.BlockSpec((1, 1), lambda i: (0, 0)),
            pl.BlockSpec((1, h), lambda i: (0, 0)),
            pl.BlockSpec((1, h), lambda i: (0, 0)),
        ],
        out_specs=(pl.BlockSpec((be, h), lambda i: (i, 0)),
                   pl.BlockSpec((be, h), lambda i: (i, 0))),
        out_shape=(jax.ShapeDtypeStruct((e, h), jnp.float32),
                   jax.ShapeDtypeStruct((e, h), jnp.float32)),
    )(gs, gd, w2e, b2e.reshape(1, h), w2c.reshape(1, h),
      b2c.reshape(1, 1), wde.reshape(1, h), wdc.reshape(1, h))


def _sc_scatter(dst_ll, m_ll, dst_rl, m_rl, zeros_hbm):
    nacc, hh = zeros_hbm.shape
    e_ll = dst_ll.shape[0]
    e_rl = dst_rl.shape[0]
    rpt = nacc // 16
    c2 = _CHUNK * 2
    mesh = plsc.VectorSubcoreMesh(core_axis_name="c", subcore_axis_name="s")

    @functools.partial(
        pl.kernel,
        out_type=jax.ShapeDtypeStruct((2, nacc, hh), jnp.float32),
        mesh=mesh,
        scratch_types=[
            pltpu.VMEM((c2,), jnp.int32),
            pltpu.VMEM((c2, hh), jnp.float32),
            pltpu.VMEM_SHARED((nacc, hh), jnp.float32),
        ],
    )
    def k(dll_h, mll_h, drl_h, mrl_h, z_h, o_h, idx_v, m_v, acc):
        cid = lax.axis_index("c")
        sid = lax.axis_index("s")
        wid = cid * 16 + sid
        r0 = sid * rpt
        pltpu.sync_copy(z_h.at[pl.ds(r0, rpt)], acc.at[pl.ds(r0, rpt)])
        plsc.subcore_barrier()

        def run(dst_h, m_h, e):
            e_per_w = e // _NW
            n_chunks = e_per_w // c2
            base_w = wid * e_per_w

            def body(i, carry):
                base = base_w + i * c2
                pltpu.sync_copy(dst_h.at[pl.ds(base, c2)], idx_v)
                pltpu.sync_copy(m_h.at[pl.ds(base, c2)], m_v)
                pltpu.sync_copy(m_v, acc.at[idx_v], add=True)
                return carry

            lax.fori_loop(0, n_chunks, body, 0)

        run(dll_h, mll_h, e_ll)
        run(drl_h, mrl_h, e_rl)
        plsc.subcore_barrier()
        pltpu.sync_copy(acc.at[pl.ds(r0, rpt)], o_h.at[cid, pl.ds(r0, rpt)])

    return k(dst_ll, m_ll, dst_rl, m_rl, zeros_hbm)


def _tc_final(h, xp, ah0, ah1, ax0, ax1, wn1a, wn1b, bn1, wn2, bn2):
    n, d = h.shape
    wx = xp.shape[1]
    bn = _row_block(n, 2048)

    def body(h_ref, xp_ref, ah0_ref, ah1_ref, ax0_ref, ax1_ref,
             wn1a_ref, wn1b_ref, bn1_ref, wn2_ref, bn2_ref, oh_ref, ox_ref):
        hv = h_ref[...]
        hn = ah0_ref[...] + ah1_ref[...]
        t = _silu(jnp.dot(hv, wn1a_ref[...], precision=_PREC,
                          preferred_element_type=jnp.float32)
                  + jnp.dot(hn, wn1b_ref[...], precision=_PREC,
                            preferred_element_type=jnp.float32)
                  + bn1_ref[...])
        oh_ref[...] = hv + jnp.dot(t, wn2_ref[...], precision=_PREC,
                                   preferred_element_type=jnp.float32) + bn2_ref[...]
        ox_ref[...] = xp_ref[...] + ax0_ref[...] + ax1_ref[...]

    return pl.pallas_call(
        body,
        grid=(n // bn,),
        in_specs=[
            pl.BlockSpec((bn, d), lambda i: (i, 0)),
            pl.BlockSpec((bn, wx), lambda i: (i, 0)),
            pl.BlockSpec((bn, d), lambda i: (i, 0)),
            pl.BlockSpec((bn, d), lambda i: (i, 0)),
            pl.BlockSpec((bn, wx), lambda i: (i, 0)),
            pl.BlockSpec((bn, wx), lambda i: (i, 0)),
            pl.BlockSpec((d, d), lambda i: (0, 0)),
            pl.BlockSpec((d, d), lambda i: (0, 0)),
            pl.BlockSpec((1, d), lambda i: (0, 0)),
            pl.BlockSpec((d, d), lambda i: (0, 0)),
            pl.BlockSpec((1, d), lambda i: (0, 0)),
        ],
        out_specs=(pl.BlockSpec((bn, d), lambda i: (i, 0)),
                   pl.BlockSpec((bn, wx), lambda i: (i, 0))),
        out_shape=(jax.ShapeDtypeStruct((n, d), jnp.float32),
                   jax.ShapeDtypeStruct((n, wx), jnp.float32)),
    )(h, xp, ah0, ah1, ax0, ax1, wn1a, wn1b, bn1.reshape(1, d), wn2,
      bn2.reshape(1, d))


def _pad_edges(src, dst, dummy):
    e = src.shape[0]
    e_pad = -(-e // _GRAN) * _GRAN
    pad = e_pad - e
    if pad:
        src = jnp.concatenate([src, jnp.zeros((pad,), jnp.int32)])
        dst = jnp.concatenate([dst, jnp.full((pad,), dummy, jnp.int32)])
    return src, dst


def kernel(h_lig, h_rec, x_lig, x_rec, edge_index_ll, edge_index_rl,
           W1e_ll, b1e_ll, W2e_ll, b2e_ll, W1c_ll, b1c_ll, W2c_ll, b2c_ll,
           W1e_rl, b1e_rl, W2e_rl, b2e_rl, W1c_rl, b1c_rl, W2c_rl, b2c_rl,
           Wn1, bn1, Wn2, bn2):
    n_lig, d = h_lig.shape

    w_lig = jnp.concatenate(
        [W1e_ll[:d], W1c_ll[:d], W1e_ll[d:2 * d], W1c_ll[d:2 * d],
         W1e_rl[d:2 * d], W1c_rl[d:2 * d]], axis=1)
    b_lig = jnp.concatenate(
        [jnp.zeros((2 * d,), jnp.float32), b1e_ll, b1c_ll, b1e_rl, b1c_rl])
    x_lig_p = jnp.pad(x_lig, ((0, 0), (0, d - x_lig.shape[1])))
    x_rec_p = jnp.pad(x_rec, ((0, 0), (0, d - x_rec.shape[1])))
    t_src_ll, t_dst_ll, t_dst_rl = _rowmm(h_lig, w_lig, b_lig, x_lig_p, 3)
    w_rec = jnp.concatenate([W1e_rl[:d], W1c_rl[:d]], axis=1)
    (t_src_rl,) = _rowmm(h_rec, w_rec, jnp.zeros((2 * d,), jnp.float32),
                         x_rec_p, 1)

    src_ll, dst_ll = _pad_edges(edge_index_ll[0], edge_index_ll[1], n_lig)
    src_rl, dst_rl = _pad_edges(edge_index_rl[0], edge_index_rl[1], n_lig)
    gs_ll, gd_ll, gs_rl, gd_rl = _sc_gather(
        t_src_ll, t_dst_ll, src_ll, dst_ll,
        t_src_rl, t_dst_rl, src_rl, dst_rl)

    mh_ll, mx_ll = _tc_edge(gs_ll, gd_ll, W2e_ll, b2e_ll, W2c_ll[:, 0], b2c_ll,
                            W1e_ll[2 * d], W1c_ll[2 * d])
    mh_rl, mx_rl = _tc_edge(gs_rl, gd_rl, W2e_rl, b2e_rl, W2c_rl[:, 0], b2c_rl,
                            W1e_rl[2 * d], W1c_rl[2 * d])

    nacc = -(-(n_lig + 1) // 128) * 128
    zh = jnp.zeros((nacc, d), jnp.float32)
    acc_h = _sc_scatter(dst_ll, mh_ll, dst_rl, mh_rl, zh)
    acc_x = _sc_scatter(dst_ll, mx_ll, dst_rl, mx_rl, zh)

    h_out, xp_out = _tc_final(
        h_lig, x_lig_p,
        acc_h[0, :n_lig], acc_h[1, :n_lig], acc_x[0, :n_lig], acc_x[1, :n_lig],
        Wn1[:d], Wn1[d:], bn1, Wn2, bn2)

    return (h_out, h_rec, xp_out[:, :x_lig.shape[1]], x_rec)


# speedup vs baseline: 1.0278x; 1.0278x over previous
import functools

import jax
import jax.numpy as jnp
from jax import lax
from jax.experimental import pallas as pl
from jax.experimental.pallas import tpu as pltpu
from jax.experimental.pallas import tpu_sc as plsc

_PREC = lax.Precision.HIGHEST
_NW = 32
_CHUNK = 64
_GRAN = _NW * _CHUNK * 2


def _silu(x):
    return x / (1.0 + jnp.exp(-x))


def _row_block(n, cap):
    b = 8
    for c in range(8, cap + 1, 8):
        if n % c == 0:
            b = c
    return b


def _rowmm(x, w, b, xpad, splits):
    n, d = x.shape
    m = w.shape[1]
    dx = xpad.shape[1]
    bn = _row_block(n, 2048)
    sw = m // splits

    def body(x_ref, w_ref, b_ref, xp_ref, *outs):
        acc = jnp.dot(x_ref[...], w_ref[...], precision=_PREC,
                      preferred_element_type=jnp.float32) + b_ref[...]
        xp = xp_ref[...]
        for j, o in enumerate(outs):
            o[...] = jnp.concatenate([acc[:, j * sw:(j + 1) * sw], xp], axis=1)

    outs = tuple(jax.ShapeDtypeStruct((n, sw + dx), jnp.float32)
                 for _ in range(splits))
    return pl.pallas_call(
        body,
        grid=(n // bn,),
        in_specs=[
            pl.BlockSpec((bn, d), lambda i: (i, 0)),
            pl.BlockSpec((d, m), lambda i: (0, 0)),
            pl.BlockSpec((1, m), lambda i: (0, 0)),
            pl.BlockSpec((bn, dx), lambda i: (i, 0)),
        ],
        out_specs=tuple(pl.BlockSpec((bn, sw + dx), lambda i: (i, 0))
                        for _ in range(splits)),
        out_shape=outs,
    )(x, w, b.reshape(1, m), xpad)


def _sc_gather(t_src_ll, t_dst_ll, src_ll, dst_ll,
               t_src_rl, t_dst_rl, src_rl, dst_rl):
    e_ll = src_ll.shape[0]
    e_rl = src_rl.shape[0]
    w = t_src_ll.shape[1]
    ei_max = max(e_ll, e_rl) // _NW
    mesh = plsc.VectorSubcoreMesh(core_axis_name="c", subcore_axis_name="s")

    @functools.partial(
        pl.kernel,
        out_type=(jax.ShapeDtypeStruct((e_ll, w), jnp.float32),
                  jax.ShapeDtypeStruct((e_ll, w), jnp.float32),
                  jax.ShapeDtypeStruct((e_rl, w), jnp.float32),
                  jax.ShapeDtypeStruct((e_rl, w), jnp.float32)),
        mesh=mesh,
        scratch_types=[
            pltpu.VMEM((ei_max,), jnp.int32),
            pltpu.VMEM((ei_max,), jnp.int32),
            pltpu.VMEM((_CHUNK, w), jnp.float32),
            pltpu.VMEM((_CHUNK, w), jnp.float32),
            pltpu.VMEM((_CHUNK, w), jnp.float32),
            pltpu.VMEM((_CHUNK, w), jnp.float32),
            pltpu.SemaphoreType.DMA,
            pltpu.SemaphoreType.DMA,
            pltpu.SemaphoreType.DMA,
            pltpu.SemaphoreType.DMA,
        ],
    )
    def k(tsll_h, tdll_h, sll_h, dll_h, tsrl_h, tdrl_h, srl_h, drl_h,
          gsll_h, gdll_h, gsrl_h, gdrl_h,
          idx_s, idx_d, rs0, rd0, rs1, rd1, sg0, sg1, sw0, sw1):
        wid = lax.axis_index("c") * 16 + lax.axis_index("s")
        slots = ((rs0, rd0, sg0, sw0), (rs1, rd1, sg1, sw1))

        def run(tsrc_h, tdst_h, src_h, dst_h, gs_h, gd_h, e):
            e_per_w = e // _NW
            n_chunks = e_per_w // _CHUNK
            base_w = wid * e_per_w
            pltpu.sync_copy(src_h.at[pl.ds(base_w, e_per_w)],
                            idx_s.at[pl.ds(0, e_per_w)])
            pltpu.sync_copy(dst_h.at[pl.ds(base_w, e_per_w)],
                            idx_d.at[pl.ds(0, e_per_w)])

            def issue(slot, i, reclaim):
                rs, rd, sg, sw = slots[slot]
                base = base_w + i * _CHUNK
                if reclaim:
                    pltpu.make_async_copy(rs, gs_h.at[pl.ds(base, _CHUNK)], sw).wait()
                    pltpu.make_async_copy(rd, gd_h.at[pl.ds(base, _CHUNK)], sw).wait()
                pltpu.async_copy(tsrc_h.at[idx_s.at[pl.ds(i * _CHUNK, _CHUNK)]], rs, sg)
                pltpu.async_copy(tdst_h.at[idx_d.at[pl.ds(i * _CHUNK, _CHUNK)]], rd, sg)

            def complete(slot, i):
                rs, rd, sg, sw = slots[slot]
                base = base_w + i * _CHUNK
                pltpu.make_async_copy(tsrc_h.at[idx_s.at[pl.ds(i * _CHUNK, _CHUNK)]], rs, sg).wait()
                pltpu.make_async_copy(tdst_h.at[idx_d.at[pl.ds(i * _CHUNK, _CHUNK)]], rd, sg).wait()
                pltpu.async_copy(rs, gs_h.at[pl.ds(base, _CHUNK)], sw)
                pltpu.async_copy(rd, gd_h.at[pl.ds(base, _CHUNK)], sw)

            issue(0, 0, False)
            issue(1, 1, False)

            def pair(p, carry):
                complete(0, 2 * p)
                issue(0, 2 * p + 2, True)
                complete(1, 2 * p + 1)
                issue(1, 2 * p + 3, True)
                return carry

            lax.fori_loop(0, n_chunks // 2 - 1, pair, 0)
            complete(0, n_chunks - 2)
            complete(1, n_chunks - 1)
            for rs, rd, sg, sw in slots:
                pltpu.make_async_copy(rs, gs_h.at[pl.ds(base_w, _CHUNK)], sw).wait()
                pltpu.make_async_copy(rd, gd_h.at[pl.ds(base_w, _CHUNK)], sw).wait()

        run(tsll_h, tdll_h, sll_h, dll_h, gsll_h, gdll_h, e_ll)
        run(tsrl_h, tdrl_h, srl_h, drl_h, gsrl_h, gdrl_h, e_rl)

    return k(t_src_ll, t_dst_ll, src_ll, dst_ll,
             t_src_rl, t_dst_rl, src_rl, dst_rl)


def _tc_edge(gs, gd, w2e, b2e, w2c, b2c, wde, wdc):
    e, w = gs.shape
    h = 128
    be = 1024

    def body(gs_ref, gd_ref, w2e_ref, b2e_ref, w2c_ref,
             b2c_ref, wde_ref, wdc_ref, oh_ref, ox_ref):
        gsv = gs_ref[...]
        gdv = gd_ref[...]
        xdiff = gsv[:, 2 * h:] - gdv[:, 2 * h:]
        d2 = jnp.sum(xdiff * xdiff, axis=1, keepdims=True)
        dij = jnp.sqrt(d2)
        xn = xdiff / (dij + 1e-9)
        ue = gsv[:, :h] + gdv[:, :h] + dij * wde_ref[...]
        uc = gsv[:, h:2 * h] + gdv[:, h:2 * h] + dij * wdc_ref[...]
        a = _silu(ue)
        mh = _silu(jnp.dot(a, w2e_ref[...], precision=_PREC,
                           preferred_element_type=jnp.float32) + b2e_ref[...])
        c = _silu(uc)
        s = _silu(jnp.sum(c * w2c_ref[...], axis=1, keepdims=True) + b2c_ref[...])
        oh_ref[...] = mh
        ox_ref[...] = s * xn

    return pl.pallas_call(
        body,
        grid=(e // be,),
        in_specs=[
            pl.BlockSpec((be, w), lambda i: (i, 0)),
            pl.BlockSpec((be, w), lambda i: (i, 0)),
            pl.BlockSpec((h, h), lambda i: (0, 0)),
            pl.BlockSpec((1, h), lambda i: (0, 0)),
            pl.BlockSpec((1, h), lambda i: (0, 0)),
            pl.BlockSpec((1, 1), lambda i: (0, 0)),
            pl.BlockSpec((1, h), lambda i: (0, 0)),
            pl.BlockSpec((1, h), lambda i: (0, 0)),
        ],
        out_specs=(pl.BlockSpec((be, h), lambda i: (i, 0)),
                   pl.BlockSpec((be, h), lambda i: (i, 0))),
        out_shape=(jax.ShapeDtypeStruct((e, h), jnp.float32),
                   jax.ShapeDtypeStruct((e, h), jnp.float32)),
    )(gs, gd, w2e, b2e.reshape(1, h), w2c.reshape(1, h),
      b2c.reshape(1, 1), wde.reshape(1, h), wdc.reshape(1, h))


def _sc_scatter(dst_ll, m_ll, dst_rl, m_rl, zeros_hbm):
    nacc, hh = zeros_hbm.shape
    e_ll = dst_ll.shape[0]
    e_rl = dst_rl.shape[0]
    rpt = nacc // 16
    c2 = _CHUNK * 2
    mesh = plsc.VectorSubcoreMesh(core_axis_name="c", subcore_axis_name="s")

    @functools.partial(
        pl.kernel,
        out_type=jax.ShapeDtypeStruct((2, nacc, hh), jnp.float32),
        mesh=mesh,
        scratch_types=[
            pltpu.VMEM((c2,), jnp.int32),
            pltpu.VMEM((c2, hh), jnp.float32),
            pltpu.VMEM_SHARED((nacc, hh), jnp.float32),
        ],
    )
    def k(dll_h, mll_h, drl_h, mrl_h, z_h, o_h, idx_v, m_v, acc):
        cid = lax.axis_index("c")
        sid = lax.axis_index("s")
        wid = cid * 16 + sid
        r0 = sid * rpt
        pltpu.sync_copy(z_h.at[pl.ds(r0, rpt)], acc.at[pl.ds(r0, rpt)])
        plsc.subcore_barrier()

        def run(dst_h, m_h, e):
            e_per_w = e // _NW
            n_chunks = e_per_w // c2
            base_w = wid * e_per_w

            def body(i, carry):
                base = base_w + i * c2
                pltpu.sync_copy(dst_h.at[pl.ds(base, c2)], idx_v)
                pltpu.sync_copy(m_h.at[pl.ds(base, c2)], m_v)
                pltpu.sync_copy(m_v, acc.at[idx_v], add=True)
                return carry

            lax.fori_loop(0, n_chunks, body, 0)

        run(dll_h, mll_h, e_ll)
        run(drl_h, mrl_h, e_rl)
        plsc.subcore_barrier()
        pltpu.sync_copy(acc.at[pl.ds(r0, rpt)], o_h.at[cid, pl.ds(r0, rpt)])

    return k(dst_ll, m_ll, dst_rl, m_rl, zeros_hbm)


def _tc_final(h, xp, ah0, ah1, ax0, ax1, wn1a, wn1b, bn1, wn2, bn2):
    n, d = h.shape
    wx = xp.shape[1]
    bn = _row_block(n, 2048)

    def body(h_ref, xp_ref, ah0_ref, ah1_ref, ax0_ref, ax1_ref,
             wn1a_ref, wn1b_ref, bn1_ref, wn2_ref, bn2_ref, oh_ref, ox_ref):
        hv = h_ref[...]
        hn = ah0_ref[...] + ah1_ref[...]
        t = _silu(jnp.dot(hv, wn1a_ref[...], precision=_PREC,
                          preferred_element_type=jnp.float32)
                  + jnp.dot(hn, wn1b_ref[...], precision=_PREC,
                            preferred_element_type=jnp.float32)
                  + bn1_ref[...])
        oh_ref[...] = hv + jnp.dot(t, wn2_ref[...], precision=_PREC,
                                   preferred_element_type=jnp.float32) + bn2_ref[...]
        ox_ref[...] = xp_ref[...] + ax0_ref[...] + ax1_ref[...]

    return pl.pallas_call(
        body,
        grid=(n // bn,),
        in_specs=[
            pl.BlockSpec((bn, d), lambda i: (i, 0)),
            pl.BlockSpec((bn, wx), lambda i: (i, 0)),
            pl.BlockSpec((bn, d), lambda i: (i, 0)),
            pl.BlockSpec((bn, d), lambda i: (i, 0)),
            pl.BlockSpec((bn, wx), lambda i: (i, 0)),
            pl.BlockSpec((bn, wx), lambda i: (i, 0)),
            pl.BlockSpec((d, d), lambda i: (0, 0)),
            pl.BlockSpec((d, d), lambda i: (0, 0)),
            pl.BlockSpec((1, d), lambda i: (0, 0)),
            pl.BlockSpec((d, d), lambda i: (0, 0)),
            pl.BlockSpec((1, d), lambda i: (0, 0)),
        ],
        out_specs=(pl.BlockSpec((bn, d), lambda i: (i, 0)),
                   pl.BlockSpec((bn, wx), lambda i: (i, 0))),
        out_shape=(jax.ShapeDtypeStruct((n, d), jnp.float32),
                   jax.ShapeDtypeStruct((n, wx), jnp.float32)),
    )(h, xp, ah0, ah1, ax0, ax1, wn1a, wn1b, bn1.reshape(1, d), wn2,
      bn2.reshape(1, d))


def _pad_edges(src, dst, dummy):
    e = src.shape[0]
    e_pad = -(-e // _GRAN) * _GRAN
    pad = e_pad - e
    if pad:
        src = jnp.concatenate([src, jnp.zeros((pad,), jnp.int32)])
        dst = jnp.concatenate([dst, jnp.full((pad,), dummy, jnp.int32)])
    return src, dst


def kernel(h_lig, h_rec, x_lig, x_rec, edge_index_ll, edge_index_rl,
           W1e_ll, b1e_ll, W2e_ll, b2e_ll, W1c_ll, b1c_ll, W2c_ll, b2c_ll,
           W1e_rl, b1e_rl, W2e_rl, b2e_rl, W1c_rl, b1c_rl, W2c_rl, b2c_rl,
           Wn1, bn1, Wn2, bn2):
    n_lig, d = h_lig.shape

    w_lig = jnp.concatenate(
        [W1e_ll[:d], W1c_ll[:d], W1e_ll[d:2 * d], W1c_ll[d:2 * d],
         W1e_rl[d:2 * d], W1c_rl[d:2 * d]], axis=1)
    b_lig = jnp.concatenate(
        [jnp.zeros((2 * d,), jnp.float32), b1e_ll, b1c_ll, b1e_rl, b1c_rl])
    x_lig_p = jnp.pad(x_lig, ((0, 0), (0, d - x_lig.shape[1])))
    x_rec_p = jnp.pad(x_rec, ((0, 0), (0, d - x_rec.shape[1])))
    t_src_ll, t_dst_ll, t_dst_rl = _rowmm(h_lig, w_lig, b_lig, x_lig_p, 3)
    w_rec = jnp.concatenate([W1e_rl[:d], W1c_rl[:d]], axis=1)
    (t_src_rl,) = _rowmm(h_rec, w_rec, jnp.zeros((2 * d,), jnp.float32),
                         x_rec_p, 1)

    src_ll, dst_ll = _pad_edges(edge_index_ll[0], edge_index_ll[1], n_lig)
    src_rl, dst_rl = _pad_edges(edge_index_rl[0], edge_index_rl[1], n_lig)
    gs_ll, gd_ll, gs_rl, gd_rl = _sc_gather(
        t_src_ll, t_dst_ll, src_ll, dst_ll,
        t_src_rl, t_dst_rl, src_rl, dst_rl)

    mh_ll, mx_ll = _tc_edge(gs_ll, gd_ll, W2e_ll, b2e_ll, W2c_ll[:, 0], b2c_ll,
                            W1e_ll[2 * d], W1c_ll[2 * d])
    mh_rl, mx_rl = _tc_edge(gs_rl, gd_rl, W2e_rl, b2e_rl, W2c_rl[:, 0], b2c_rl,
                            W1e_rl[2 * d], W1c_rl[2 * d])

    nacc = -(-(n_lig + 1) // 128) * 128
    zh = jnp.zeros((nacc, d), jnp.float32)
    acc_h = _sc_scatter(dst_ll, mh_ll, dst_rl, mh_rl, zh)
    acc_x = _sc_scatter(dst_ll, mx_ll, dst_rl, mx_rl, zh)

    h_out, xp_out = _tc_final(
        h_lig, x_lig_p,
        acc_h[0, :n_lig], acc_h[1, :n_lig], acc_x[0, :n_lig], acc_x[1, :n_lig],
        Wn1[:d], Wn1[d:], bn1, Wn2, bn2)

    return (h_out, h_rec, xp_out[:, :x_lig.shape[1]], x_rec)


# speedup vs baseline: 1.3696x; 1.3325x over previous
import functools

import jax
import jax.numpy as jnp
from jax import lax
from jax.experimental import pallas as pl
from jax.experimental.pallas import tpu as pltpu
from jax.experimental.pallas import tpu_sc as plsc

_PREC = lax.Precision.HIGHEST
_NW = 32
_CHUNK = 64
_GRAN = _NW * _CHUNK * 2


def _silu(x):
    return x / (1.0 + jnp.exp(-x))


def _row_block(n, cap):
    b = 8
    for c in range(8, cap + 1, 8):
        if n % c == 0:
            b = c
    return b


def _bf16_hi(x):
    b = jax.lax.bitcast_convert_type(x, jnp.uint32)
    r = b + jnp.uint32(0x7FFF) + ((b >> 16) & jnp.uint32(1))
    return r & jnp.uint32(0xFFFF0000)


def _rowmm(x, w, b, splits):
    n, d = x.shape
    m = w.shape[1]
    bn = _row_block(n, 2048)
    sw = m // splits

    def body(x_ref, w_ref, b_ref, *outs):
        acc = jnp.dot(x_ref[...], w_ref[...], precision=_PREC,
                      preferred_element_type=jnp.float32) + b_ref[...]
        for j, o in enumerate(outs):
            hi = _bf16_hi(acc[:, j * sw:j * sw + 128])
            lo = _bf16_hi(acc[:, j * sw + 128:j * sw + 256]) >> 16
            o[...] = jax.lax.bitcast_convert_type(hi | lo, jnp.int32)

    outs = tuple(jax.ShapeDtypeStruct((n, 128), jnp.int32)
                 for _ in range(splits))
    return pl.pallas_call(
        body,
        grid=(n // bn,),
        in_specs=[
            pl.BlockSpec((bn, d), lambda i: (i, 0)),
            pl.BlockSpec((d, m), lambda i: (0, 0)),
            pl.BlockSpec((1, m), lambda i: (0, 0)),
        ],
        out_specs=tuple(pl.BlockSpec((bn, 128), lambda i: (i, 0))
                        for _ in range(splits)),
        out_shape=outs,
    )(x, w, b.reshape(1, m))


def _sc_gather(t_src_ll, t_dst_ll, x_src_ll, x_dst_ll, src_ll, dst_ll,
               t_src_rl, t_dst_rl, x_src_rl, x_dst_rl, src_rl, dst_rl):
    e_ll = src_ll.shape[0]
    e_rl = src_rl.shape[0]
    dx = x_src_ll.shape[1]
    ei_max = max(e_ll, e_rl) // _NW
    mesh = plsc.VectorSubcoreMesh(core_axis_name="c", subcore_axis_name="s")

    @functools.partial(
        pl.kernel,
        out_type=(jax.ShapeDtypeStruct((e_ll, 128), jnp.int32),
                  jax.ShapeDtypeStruct((e_ll, 128), jnp.int32),
                  jax.ShapeDtypeStruct((e_ll, dx), jnp.float32),
                  jax.ShapeDtypeStruct((e_ll, dx), jnp.float32),
                  jax.ShapeDtypeStruct((e_rl, 128), jnp.int32),
                  jax.ShapeDtypeStruct((e_rl, 128), jnp.int32),
                  jax.ShapeDtypeStruct((e_rl, dx), jnp.float32),
                  jax.ShapeDtypeStruct((e_rl, dx), jnp.float32)),
        mesh=mesh,
        scratch_types=[
            pltpu.VMEM((ei_max,), jnp.int32),
            pltpu.VMEM((ei_max,), jnp.int32),
            pltpu.VMEM((_CHUNK, 128), jnp.int32),
            pltpu.VMEM((_CHUNK, 128), jnp.int32),
            pltpu.VMEM((_CHUNK, dx), jnp.float32),
            pltpu.VMEM((_CHUNK, dx), jnp.float32),
            pltpu.VMEM((_CHUNK, 128), jnp.int32),
            pltpu.VMEM((_CHUNK, 128), jnp.int32),
            pltpu.VMEM((_CHUNK, dx), jnp.float32),
            pltpu.VMEM((_CHUNK, dx), jnp.float32),
            pltpu.SemaphoreType.DMA,
            pltpu.SemaphoreType.DMA,
            pltpu.SemaphoreType.DMA,
            pltpu.SemaphoreType.DMA,
        ],
    )
    def k(tsll_h, tdll_h, xsll_h, xdll_h, sll_h, dll_h,
          tsrl_h, tdrl_h, xsrl_h, xdrl_h, srl_h, drl_h,
          gsll_h, gdll_h, hxsll_h, hxdll_h, gsrl_h, gdrl_h, hxsrl_h, hxdrl_h,
          idx_s, idx_d, ts0, td0, xs0, xd0, ts1, td1, xs1, xd1,
          sg0, sg1, sw0, sw1):
        wid = lax.axis_index("c") * 16 + lax.axis_index("s")
        slots = (((ts0, td0, xs0, xd0), sg0, sw0),
                 ((ts1, td1, xs1, xd1), sg1, sw1))

        def run(tsrc_h, tdst_h, xsrc_h, xdst_h, src_h, dst_h,
                gs_h, gd_h, hxs_h, hxd_h, e):
            e_per_w = e // _NW
            n_chunks = e_per_w // _CHUNK
            base_w = wid * e_per_w
            pltpu.sync_copy(src_h.at[pl.ds(base_w, e_per_w)],
                            idx_s.at[pl.ds(0, e_per_w)])
            pltpu.sync_copy(dst_h.at[pl.ds(base_w, e_per_w)],
                            idx_d.at[pl.ds(0, e_per_w)])

            def hbm_slices(i):
                base = base_w + i * _CHUNK
                return (gs_h.at[pl.ds(base, _CHUNK)],
                        gd_h.at[pl.ds(base, _CHUNK)],
                        hxs_h.at[pl.ds(base, _CHUNK)],
                        hxd_h.at[pl.ds(base, _CHUNK)])

            def tables(i):
                isl = idx_s.at[pl.ds(i * _CHUNK, _CHUNK)]
                idl = idx_d.at[pl.ds(i * _CHUNK, _CHUNK)]
                return (tsrc_h.at[isl], tdst_h.at[idl],
                        xsrc_h.at[isl], xdst_h.at[idl])

            def issue(slot, i, reclaim):
                bufs, sg, sw = slots[slot]
                outs = hbm_slices(i)
                if reclaim:
                    for b, o in zip(bufs, outs):
                        pltpu.make_async_copy(b, o, sw).wait()
                for t, b in zip(tables(i), bufs):
                    pltpu.async_copy(t, b, sg)

            def complete(slot, i):
                bufs, sg, sw = slots[slot]
                for t, b in zip(tables(i), bufs):
                    pltpu.make_async_copy(t, b, sg).wait()
                for b, o in zip(bufs, hbm_slices(i)):
                    pltpu.async_copy(b, o, sw)

            issue(0, 0, False)
            issue(1, 1, False)

            def pair(p, carry):
                complete(0, 2 * p)
                issue(0, 2 * p + 2, True)
                complete(1, 2 * p + 1)
                issue(1, 2 * p + 3, True)
                return carry

            lax.fori_loop(0, n_chunks // 2 - 1, pair, 0)
            complete(0, n_chunks - 2)
            complete(1, n_chunks - 1)
            for bufs, sg, sw in slots:
                for b, o in zip(bufs, hbm_slices(0)):
                    pltpu.make_async_copy(b, o, sw).wait()

        run(tsll_h, tdll_h, xsll_h, xdll_h, sll_h, dll_h,
            gsll_h, gdll_h, hxsll_h, hxdll_h, e_ll)
        run(tsrl_h, tdrl_h, xsrl_h, xdrl_h, srl_h, drl_h,
            gsrl_h, gdrl_h, hxsrl_h, hxdrl_h, e_rl)

    return k(t_src_ll, t_dst_ll, x_src_ll, x_dst_ll, src_ll, dst_ll,
             t_src_rl, t_dst_rl, x_src_rl, x_dst_rl, src_rl, dst_rl)


def _tc_edge(gs, gd, xs, xd, w2e, b2e, w2c, b2c, wde, wdc):
    e = gs.shape[0]
    h = 128
    be = 1024

    def unpack(v):
        u = jax.lax.bitcast_convert_type(v, jnp.uint32)
        hi = jax.lax.bitcast_convert_type(u & jnp.uint32(0xFFFF0000),
                                          jnp.float32)
        lo = jax.lax.bitcast_convert_type(u << 16, jnp.float32)
        return hi, lo

    def body(gs_ref, gd_ref, xs_ref, xd_ref, w2e_ref, b2e_ref, w2c_ref,
             b2c_ref, wde_ref, wdc_ref, oh_ref, ox_ref):
        xsv = xs_ref[...]
        xdv = xd_ref[...]
        xdiff = xsv - xdv
        d2 = jnp.sum(xdiff * xdiff, axis=1, keepdims=True)
        dij = jnp.sqrt(d2)
        xn = xdiff / (dij + 1e-9)
        se, sc = unpack(gs_ref[...])
        de, dc = unpack(gd_ref[...])
        ue = se + de + dij * wde_ref[...]
        uc = sc + dc + dij * wdc_ref[...]
        a = _silu(ue).astype(jnp.bfloat16)
        mh = _silu(jnp.dot(a, w2e_ref[...].astype(jnp.bfloat16),
                           preferred_element_type=jnp.float32) + b2e_ref[...])
        c = _silu(uc)
        s = _silu(jnp.sum(c * w2c_ref[...], axis=1, keepdims=True) + b2c_ref[...])
        oh_ref[...] = mh
        ox_ref[...] = s * xn

    return pl.pallas_call(
        body,
        grid=(e // be,),
        in_specs=[
            pl.BlockSpec((be, h), lambda i: (i, 0)),
            pl.BlockSpec((be, h), lambda i: (i, 0)),
            pl.BlockSpec((be, h), lambda i: (i, 0)),
            pl.BlockSpec((be, h), lambda i: (i, 0)),
            pl.BlockSpec((h, h), lambda i: (0, 0)),
            pl.BlockSpec((1, h), lambda i: (0, 0)),
            pl.BlockSpec((1, h), lambda i: (0, 0)),
            pl.BlockSpec((1, 1), lambda i: (0, 0)),
            pl.BlockSpec((1, h), lambda i: (0, 0)),
            pl.BlockSpec((1, h), lambda i: (0, 0)),
        ],
        out_specs=(pl.BlockSpec((be, h), lambda i: (i, 0)),
                   pl.BlockSpec((be, h), lambda i: (i, 0))),
        out_shape=(jax.ShapeDtypeStruct((e, h), jnp.float32),
                   jax.ShapeDtypeStruct((e, h), jnp.float32)),
    )(gs, gd, xs, xd, w2e, b2e.reshape(1, h), w2c.reshape(1, h),
      b2c.reshape(1, 1), wde.reshape(1, h), wdc.reshape(1, h))


def _sc_scatter(dst_ll, m_ll, dst_rl, m_rl, zeros_hbm):
    nacc, hh = zeros_hbm.shape
    c2 = _CHUNK * 2
    e_ll = dst_ll.shape[0]
    e_rl = dst_rl.shape[0]
    rpt = nacc // 16
    mesh = plsc.VectorSubcoreMesh(core_axis_name="c", subcore_axis_name="s")

    @functools.partial(
        pl.kernel,
        out_type=jax.ShapeDtypeStruct((2, nacc, hh), jnp.float32),
        mesh=mesh,
        scratch_types=[
            pltpu.VMEM((c2,), jnp.int32),
            pltpu.VMEM((c2,), jnp.int32),
            pltpu.VMEM((c2, hh), jnp.float32),
            pltpu.VMEM((c2, hh), jnp.float32),
            pltpu.VMEM_SHARED((nacc, hh), jnp.float32),
            pltpu.SemaphoreType.DMA,
            pltpu.SemaphoreType.DMA,
        ],
    )
    def k(dll_h, mll_h, drl_h, mrl_h, z_h, o_h, i0, i1, m0, m1, acc, sl0, sl1):
        cid = lax.axis_index("c")
        sid = lax.axis_index("s")
        wid = cid * 16 + sid
        r0 = sid * rpt
        pltpu.sync_copy(z_h.at[pl.ds(r0, rpt)], acc.at[pl.ds(r0, rpt)])
        plsc.subcore_barrier()
        slots = ((m0, i0, sl0), (m1, i1, sl1))

        def run(dst_h, m_h, e):
            e_per_w = e // _NW
            n_chunks = e_per_w // c2
            base_w = wid * e_per_w

            def load(slot, i):
                m_v, i_v, sem = slots[slot]
                base = base_w + i * c2
                pltpu.async_copy(dst_h.at[pl.ds(base, c2)], i_v, sem)
                pltpu.async_copy(m_h.at[pl.ds(base, c2)], m_v, sem)

            def scat(slot, i):
                m_v, i_v, sem = slots[slot]
                base = base_w + i * c2
                pltpu.make_async_copy(dst_h.at[pl.ds(base, c2)], i_v, sem).wait()
                pltpu.make_async_copy(m_h.at[pl.ds(base, c2)], m_v, sem).wait()
                pltpu.sync_copy(m_v, acc.at[i_v], add=True)

            load(0, 0)
            load(1, 1)

            def pair(p, carry):
                scat(0, 2 * p)
                load(0, 2 * p + 2)
                scat(1, 2 * p + 1)
                load(1, 2 * p + 3)
                return carry

            lax.fori_loop(0, n_chunks // 2 - 1, pair, 0)
            scat(0, n_chunks - 2)
            scat(1, n_chunks - 1)

        run(dll_h, mll_h, e_ll)
        run(drl_h, mrl_h, e_rl)
        plsc.subcore_barrier()
        pltpu.sync_copy(acc.at[pl.ds(r0, rpt)], o_h.at[cid, pl.ds(r0, rpt)])

    return k(dst_ll, m_ll, dst_rl, m_rl, zeros_hbm)


def _tc_final(h, xp, ah0, ah1, ax0, ax1, wn1a, wn1b, bn1, wn2, bn2):
    n, d = h.shape
    wx = xp.shape[1]
    bn = _row_block(n, 2048)

    def body(h_ref, xp_ref, ah0_ref, ah1_ref, ax0_ref, ax1_ref,
             wn1a_ref, wn1b_ref, bn1_ref, wn2_ref, bn2_ref, oh_ref, ox_ref):
        hv = h_ref[...]
        hn = ah0_ref[...] + ah1_ref[...]
        t = _silu(jnp.dot(hv, wn1a_ref[...], precision=_PREC,
                          preferred_element_type=jnp.float32)
                  + jnp.dot(hn, wn1b_ref[...], precision=_PREC,
                            preferred_element_type=jnp.float32)
                  + bn1_ref[...])
        oh_ref[...] = hv + jnp.dot(t, wn2_ref[...], precision=_PREC,
                                   preferred_element_type=jnp.float32) + bn2_ref[...]
        ox_ref[...] = xp_ref[...] + ax0_ref[...] + ax1_ref[...]

    return pl.pallas_call(
        body,
        grid=(n // bn,),
        in_specs=[
            pl.BlockSpec((bn, d), lambda i: (i, 0)),
            pl.BlockSpec((bn, wx), lambda i: (i, 0)),
            pl.BlockSpec((bn, d), lambda i: (i, 0)),
            pl.BlockSpec((bn, d), lambda i: (i, 0)),
            pl.BlockSpec((bn, wx), lambda i: (i, 0)),
            pl.BlockSpec((bn, wx), lambda i: (i, 0)),
            pl.BlockSpec((d, d), lambda i: (0, 0)),
            pl.BlockSpec((d, d), lambda i: (0, 0)),
            pl.BlockSpec((1, d), lambda i: (0, 0)),
            pl.BlockSpec((d, d), lambda i: (0, 0)),
            pl.BlockSpec((1, d), lambda i: (0, 0)),
        ],
        out_specs=(pl.BlockSpec((bn, d), lambda i: (i, 0)),
                   pl.BlockSpec((bn, wx), lambda i: (i, 0))),
        out_shape=(jax.ShapeDtypeStruct((n, d), jnp.float32),
                   jax.ShapeDtypeStruct((n, wx), jnp.float32)),
    )(h, xp, ah0, ah1, ax0, ax1, wn1a, wn1b, bn1.reshape(1, d), wn2,
      bn2.reshape(1, d))


def _pad_edges(src, dst, dummy):
    e = src.shape[0]
    e_pad = -(-e // _GRAN) * _GRAN
    pad = e_pad - e
    if pad:
        src = jnp.concatenate([src, jnp.zeros((pad,), jnp.int32)])
        dst = jnp.concatenate([dst, jnp.full((pad,), dummy, jnp.int32)])
    return src, dst


def kernel(h_lig, h_rec, x_lig, x_rec, edge_index_ll, edge_index_rl,
           W1e_ll, b1e_ll, W2e_ll, b2e_ll, W1c_ll, b1c_ll, W2c_ll, b2c_ll,
           W1e_rl, b1e_rl, W2e_rl, b2e_rl, W1c_rl, b1c_rl, W2c_rl, b2c_rl,
           Wn1, bn1, Wn2, bn2):
    n_lig, d = h_lig.shape

    w_lig = jnp.concatenate(
        [W1e_ll[:d], W1c_ll[:d], W1e_ll[d:2 * d], W1c_ll[d:2 * d],
         W1e_rl[d:2 * d], W1c_rl[d:2 * d]], axis=1)
    b_lig = jnp.concatenate(
        [jnp.zeros((2 * d,), jnp.float32), b1e_ll, b1c_ll, b1e_rl, b1c_rl])
    x_lig_p = jnp.pad(x_lig, ((0, 0), (0, d - x_lig.shape[1])))
    x_rec_p = jnp.pad(x_rec, ((0, 0), (0, d - x_rec.shape[1])))
    t_src_ll, t_dst_ll, t_dst_rl = _rowmm(h_lig, w_lig, b_lig, 3)
    w_rec = jnp.concatenate([W1e_rl[:d], W1c_rl[:d]], axis=1)
    (t_src_rl,) = _rowmm(h_rec, w_rec, jnp.zeros((2 * d,), jnp.float32), 1)

    src_ll, dst_ll = _pad_edges(edge_index_ll[0], edge_index_ll[1], n_lig)
    src_rl, dst_rl = _pad_edges(edge_index_rl[0], edge_index_rl[1], n_lig)
    (gs_ll, gd_ll, xs_ll, xd_ll,
     gs_rl, gd_rl, xs_rl, xd_rl) = _sc_gather(
        t_src_ll, t_dst_ll, x_lig_p, x_lig_p, src_ll, dst_ll,
        t_src_rl, t_dst_rl, x_rec_p, x_lig_p, src_rl, dst_rl)

    mh_ll, mx_ll = _tc_edge(gs_ll, gd_ll, xs_ll, xd_ll,
                            W2e_ll, b2e_ll, W2c_ll[:, 0], b2c_ll,
                            W1e_ll[2 * d], W1c_ll[2 * d])
    mh_rl, mx_rl = _tc_edge(gs_rl, gd_rl, xs_rl, xd_rl,
                            W2e_rl, b2e_rl, W2c_rl[:, 0], b2c_rl,
                            W1e_rl[2 * d], W1c_rl[2 * d])

    nacc = -(-(n_lig + 1) // 128) * 128
    zh = jnp.zeros((nacc, d), jnp.float32)
    acc_h = _sc_scatter(dst_ll, mh_ll, dst_rl, mh_rl, zh)
    acc_x = _sc_scatter(dst_ll, mx_ll, dst_rl, mx_rl, zh)

    h_out, xp_out = _tc_final(
        h_lig, x_lig_p,
        acc_h[0, :n_lig], acc_h[1, :n_lig], acc_x[0, :n_lig], acc_x[1, :n_lig],
        Wn1[:d], Wn1[d:], bn1, Wn2, bn2)

    return (h_out, h_rec, xp_out[:, :x_lig.shape[1]], x_rec)


# speedup vs baseline: 1.4207x; 1.0373x over previous
import functools

import jax
import jax.numpy as jnp
from jax import lax
from jax.experimental import pallas as pl
from jax.experimental.pallas import tpu as pltpu
from jax.experimental.pallas import tpu_sc as plsc

_PREC = lax.Precision.HIGHEST
_NW = 32
_CHUNK = 64
_GRAN = _NW * _CHUNK * 2


def _silu(x):
    return x / (1.0 + jnp.exp(-x))


def _row_block(n, cap):
    b = 8
    for c in range(8, cap + 1, 8):
        if n % c == 0:
            b = c
    return b


def _bf16_hi(x):
    b = jax.lax.bitcast_convert_type(x, jnp.uint32)
    r = b + jnp.uint32(0x7FFF) + ((b >> 16) & jnp.uint32(1))
    return r & jnp.uint32(0xFFFF0000)


def _rowmm(x, w, b, splits):
    n, d = x.shape
    m = w.shape[1]
    bn = _row_block(n, 2048)
    sw = m // splits

    def body(x_ref, w_ref, b_ref, *outs):
        acc = jnp.dot(x_ref[...], w_ref[...], precision=_PREC,
                      preferred_element_type=jnp.float32) + b_ref[...]
        for j, o in enumerate(outs):
            hi = _bf16_hi(acc[:, j * sw:j * sw + 128])
            lo = _bf16_hi(acc[:, j * sw + 128:j * sw + 256]) >> 16
            o[...] = jax.lax.bitcast_convert_type(hi | lo, jnp.int32)

    outs = tuple(jax.ShapeDtypeStruct((n, 128), jnp.int32)
                 for _ in range(splits))
    return pl.pallas_call(
        body,
        grid=(n // bn,),
        in_specs=[
            pl.BlockSpec((bn, d), lambda i: (i, 0)),
            pl.BlockSpec((d, m), lambda i: (0, 0)),
            pl.BlockSpec((1, m), lambda i: (0, 0)),
        ],
        out_specs=tuple(pl.BlockSpec((bn, 128), lambda i: (i, 0))
                        for _ in range(splits)),
        out_shape=outs,
    )(x, w, b.reshape(1, m))


def _sc_gather(t_src_ll, t_dst_ll, x_src_ll, x_dst_ll, src_ll, dst_ll,
               t_src_rl, t_dst_rl, x_src_rl, x_dst_rl, src_rl, dst_rl):
    e_ll = src_ll.shape[0]
    e_rl = src_rl.shape[0]
    dx = x_src_ll.shape[1]
    ei_max = max(e_ll, e_rl) // _NW
    mesh = plsc.VectorSubcoreMesh(core_axis_name="c", subcore_axis_name="s")

    @functools.partial(
        pl.kernel,
        out_type=(jax.ShapeDtypeStruct((e_ll, 128), jnp.int32),
                  jax.ShapeDtypeStruct((e_ll, 128), jnp.int32),
                  jax.ShapeDtypeStruct((e_ll, dx), jnp.float32),
                  jax.ShapeDtypeStruct((e_rl, 128), jnp.int32),
                  jax.ShapeDtypeStruct((e_rl, 128), jnp.int32),
                  jax.ShapeDtypeStruct((e_rl, dx), jnp.float32)),
        mesh=mesh,
        scratch_types=[
            pltpu.VMEM((ei_max,), jnp.int32),
            pltpu.VMEM((ei_max,), jnp.int32),
            pltpu.VMEM((_CHUNK, 128), jnp.int32),
            pltpu.VMEM((_CHUNK, 128), jnp.int32),
            pltpu.VMEM((_CHUNK, dx), jnp.float32),
            pltpu.VMEM((_CHUNK, dx), jnp.float32),
            pltpu.VMEM((_CHUNK, 128), jnp.int32),
            pltpu.VMEM((_CHUNK, 128), jnp.int32),
            pltpu.VMEM((_CHUNK, dx), jnp.float32),
            pltpu.VMEM((_CHUNK, dx), jnp.float32),
            pltpu.SemaphoreType.DMA,
            pltpu.SemaphoreType.DMA,
            pltpu.SemaphoreType.DMA,
            pltpu.SemaphoreType.DMA,
        ],
    )
    def k(tsll_h, tdll_h, xsll_h, xdll_h, sll_h, dll_h,
          tsrl_h, tdrl_h, xsrl_h, xdrl_h, srl_h, drl_h,
          gsll_h, gdll_h, xfll_h, gsrl_h, gdrl_h, xfrl_h,
          idx_s, idx_d, ts0, td0, xs0, xd0, ts1, td1, xs1, xd1,
          sg0, sg1, sw0, sw1):
        wid = lax.axis_index("c") * 16 + lax.axis_index("s")
        slots = (((ts0, td0, xs0, xd0), sg0, sw0),
                 ((ts1, td1, xs1, xd1), sg1, sw1))

        def run(tsrc_h, tdst_h, xsrc_h, xdst_h, src_h, dst_h,
                gs_h, gd_h, xf_h, e):
            e_per_w = e // _NW
            n_chunks = e_per_w // _CHUNK
            base_w = wid * e_per_w
            pltpu.sync_copy(src_h.at[pl.ds(base_w, e_per_w)],
                            idx_s.at[pl.ds(0, e_per_w)])
            pltpu.sync_copy(dst_h.at[pl.ds(base_w, e_per_w)],
                            idx_d.at[pl.ds(0, e_per_w)])

            def out_slices(i):
                base = base_w + i * _CHUNK
                return (gs_h.at[pl.ds(base, _CHUNK)],
                        gd_h.at[pl.ds(base, _CHUNK)],
                        xf_h.at[pl.ds(base, _CHUNK)])

            def tables(i):
                isl = idx_s.at[pl.ds(i * _CHUNK, _CHUNK)]
                idl = idx_d.at[pl.ds(i * _CHUNK, _CHUNK)]
                return (tsrc_h.at[isl], tdst_h.at[idl],
                        xsrc_h.at[isl], xdst_h.at[idl])

            def issue(slot, i, reclaim):
                bufs, sg, sw = slots[slot]
                if reclaim:
                    for b, o in zip(bufs[:3], out_slices(i)):
                        pltpu.make_async_copy(b, o, sw).wait()
                for t, b in zip(tables(i), bufs):
                    pltpu.async_copy(t, b, sg)

            def complete(slot, i):
                bufs, sg, sw = slots[slot]
                for t, b in zip(tables(i), bufs):
                    pltpu.make_async_copy(t, b, sg).wait()
                xs_b, xd_b = bufs[2], bufs[3]

                def vrow(r, carry):
                    for g in range(dx // 16):
                        xs_b[r, pl.ds(g * 16, 16)] = (
                            xs_b[r, pl.ds(g * 16, 16)]
                            - xd_b[r, pl.ds(g * 16, 16)])
                    return carry

                lax.fori_loop(0, _CHUNK, vrow, 0)
                for b, o in zip(bufs[:3], out_slices(i)):
                    pltpu.async_copy(b, o, sw)

            issue(0, 0, False)
            issue(1, 1, False)

            def pair(p, carry):
                complete(0, 2 * p)
                issue(0, 2 * p + 2, True)
                complete(1, 2 * p + 1)
                issue(1, 2 * p + 3, True)
                return carry

            lax.fori_loop(0, n_chunks // 2 - 1, pair, 0)
            complete(0, n_chunks - 2)
            complete(1, n_chunks - 1)
            for bufs, sg, sw in slots:
                for b, o in zip(bufs[:3], out_slices(0)):
                    pltpu.make_async_copy(b, o, sw).wait()

        run(tsll_h, tdll_h, xsll_h, xdll_h, sll_h, dll_h,
            gsll_h, gdll_h, xfll_h, e_ll)
        run(tsrl_h, tdrl_h, xsrl_h, xdrl_h, srl_h, drl_h,
            gsrl_h, gdrl_h, xfrl_h, e_rl)

    return k(t_src_ll, t_dst_ll, x_src_ll, x_dst_ll, src_ll, dst_ll,
             t_src_rl, t_dst_rl, x_src_rl, x_dst_rl, src_rl, dst_rl)


def _tc_edge(gs, gd, xdf, w2e, b2e, w2c, b2c, wde, wdc):
    e = gs.shape[0]
    h = 128
    be = 1024

    def unpack(v):
        u = jax.lax.bitcast_convert_type(v, jnp.uint32)
        hi = jax.lax.bitcast_convert_type(u & jnp.uint32(0xFFFF0000),
                                          jnp.float32)
        lo = jax.lax.bitcast_convert_type(u << 16, jnp.float32)
        return hi, lo

    def body(gs_ref, gd_ref, xdf_ref, w2e_ref, b2e_ref, w2c_ref,
             b2c_ref, wde_ref, wdc_ref, oh_ref, ox_ref):
        xdiff = xdf_ref[...]
        d2 = jnp.sum(xdiff * xdiff, axis=1, keepdims=True)
        dij = jnp.sqrt(d2)
        xn = xdiff / (dij + 1e-9)
        se, sc = unpack(gs_ref[...])
        de, dc = unpack(gd_ref[...])
        ue = se + de + dij * wde_ref[...]
        uc = sc + dc + dij * wdc_ref[...]
        a = _silu(ue).astype(jnp.bfloat16)
        mh = _silu(jnp.dot(a, w2e_ref[...].astype(jnp.bfloat16),
                           preferred_element_type=jnp.float32) + b2e_ref[...])
        c = _silu(uc)
        s = _silu(jnp.sum(c * w2c_ref[...], axis=1, keepdims=True) + b2c_ref[...])
        oh_ref[...] = mh
        ox_ref[...] = s * xn

    return pl.pallas_call(
        body,
        grid=(e // be,),
        in_specs=[
            pl.BlockSpec((be, h), lambda i: (i, 0)),
            pl.BlockSpec((be, h), lambda i: (i, 0)),
            pl.BlockSpec((be, h), lambda i: (i, 0)),
            pl.BlockSpec((h, h), lambda i: (0, 0)),
            pl.BlockSpec((1, h), lambda i: (0, 0)),
            pl.BlockSpec((1, h), lambda i: (0, 0)),
            pl.BlockSpec((1, 1), lambda i: (0, 0)),
            pl.BlockSpec((1, h), lambda i: (0, 0)),
            pl.BlockSpec((1, h), lambda i: (0, 0)),
        ],
        out_specs=(pl.BlockSpec((be, h), lambda i: (i, 0)),
                   pl.BlockSpec((be, h), lambda i: (i, 0))),
        out_shape=(jax.ShapeDtypeStruct((e, h), jnp.float32),
                   jax.ShapeDtypeStruct((e, h), jnp.float32)),
    )(gs, gd, xdf, w2e, b2e.reshape(1, h), w2c.reshape(1, h),
      b2c.reshape(1, 1), wde.reshape(1, h), wdc.reshape(1, h))


def _sc_scatter(dst_ll, m_ll, dst_rl, m_rl, zeros_hbm):
    nacc, hh = zeros_hbm.shape
    c2 = _CHUNK * 2
    e_ll = dst_ll.shape[0]
    e_rl = dst_rl.shape[0]
    rpt = nacc // 16
    mesh = plsc.VectorSubcoreMesh(core_axis_name="c", subcore_axis_name="s")

    @functools.partial(
        pl.kernel,
        out_type=jax.ShapeDtypeStruct((2, nacc, hh), jnp.float32),
        mesh=mesh,
        scratch_types=[
            pltpu.VMEM((c2,), jnp.int32),
            pltpu.VMEM((c2,), jnp.int32),
            pltpu.VMEM((c2, hh), jnp.float32),
            pltpu.VMEM((c2, hh), jnp.float32),
            pltpu.VMEM_SHARED((nacc, hh), jnp.float32),
            pltpu.SemaphoreType.DMA,
            pltpu.SemaphoreType.DMA,
        ],
    )
    def k(dll_h, mll_h, drl_h, mrl_h, z_h, o_h, i0, i1, m0, m1, acc, sl0, sl1):
        cid = lax.axis_index("c")
        sid = lax.axis_index("s")
        wid = cid * 16 + sid
        r0 = sid * rpt
        pltpu.sync_copy(z_h.at[pl.ds(r0, rpt)], acc.at[pl.ds(r0, rpt)])
        plsc.subcore_barrier()
        slots = ((m0, i0, sl0), (m1, i1, sl1))

        def run(dst_h, m_h, e):
            e_per_w = e // _NW
            n_chunks = e_per_w // c2
            base_w = wid * e_per_w

            def load(slot, i):
                m_v, i_v, sem = slots[slot]
                base = base_w + i * c2
                pltpu.async_copy(dst_h.at[pl.ds(base, c2)], i_v, sem)
                pltpu.async_copy(m_h.at[pl.ds(base, c2)], m_v, sem)

            def scat(slot, i):
                m_v, i_v, sem = slots[slot]
                base = base_w + i * c2
                pltpu.make_async_copy(dst_h.at[pl.ds(base, c2)], i_v, sem).wait()
                pltpu.make_async_copy(m_h.at[pl.ds(base, c2)], m_v, sem).wait()
                pltpu.sync_copy(m_v, acc.at[i_v], add=True)

            load(0, 0)
            load(1, 1)

            def pair(p, carry):
                scat(0, 2 * p)
                load(0, 2 * p + 2)
                scat(1, 2 * p + 1)
                load(1, 2 * p + 3)
                return carry

            lax.fori_loop(0, n_chunks // 2 - 1, pair, 0)
            scat(0, n_chunks - 2)
            scat(1, n_chunks - 1)

        run(dll_h, mll_h, e_ll)
        run(drl_h, mrl_h, e_rl)
        plsc.subcore_barrier()
        pltpu.sync_copy(acc.at[pl.ds(r0, rpt)], o_h.at[cid, pl.ds(r0, rpt)])

    return k(dst_ll, m_ll, dst_rl, m_rl, zeros_hbm)


def _tc_final(h, xp, ah0, ah1, ax0, ax1, wn1a, wn1b, bn1, wn2, bn2):
    n, d = h.shape
    wx = xp.shape[1]
    bn = _row_block(n, 2048)

    def body(h_ref, xp_ref, ah0_ref, ah1_ref, ax0_ref, ax1_ref,
             wn1a_ref, wn1b_ref, bn1_ref, wn2_ref, bn2_ref, oh_ref, ox_ref):
        hv = h_ref[...]
        hn = ah0_ref[...] + ah1_ref[...]
        t = _silu(jnp.dot(hv, wn1a_ref[...], precision=_PREC,
                          preferred_element_type=jnp.float32)
                  + jnp.dot(hn, wn1b_ref[...], precision=_PREC,
                            preferred_element_type=jnp.float32)
                  + bn1_ref[...])
        oh_ref[...] = hv + jnp.dot(t, wn2_ref[...], precision=_PREC,
                                   preferred_element_type=jnp.float32) + bn2_ref[...]
        ox_ref[...] = xp_ref[...] + ax0_ref[...] + ax1_ref[...]

    return pl.pallas_call(
        body,
        grid=(n // bn,),
        in_specs=[
            pl.BlockSpec((bn, d), lambda i: (i, 0)),
            pl.BlockSpec((bn, wx), lambda i: (i, 0)),
            pl.BlockSpec((bn, d), lambda i: (i, 0)),
            pl.BlockSpec((bn, d), lambda i: (i, 0)),
            pl.BlockSpec((bn, wx), lambda i: (i, 0)),
            pl.BlockSpec((bn, wx), lambda i: (i, 0)),
            pl.BlockSpec((d, d), lambda i: (0, 0)),
            pl.BlockSpec((d, d), lambda i: (0, 0)),
            pl.BlockSpec((1, d), lambda i: (0, 0)),
            pl.BlockSpec((d, d), lambda i: (0, 0)),
            pl.BlockSpec((1, d), lambda i: (0, 0)),
        ],
        out_specs=(pl.BlockSpec((bn, d), lambda i: (i, 0)),
                   pl.BlockSpec((bn, wx), lambda i: (i, 0))),
        out_shape=(jax.ShapeDtypeStruct((n, d), jnp.float32),
                   jax.ShapeDtypeStruct((n, wx), jnp.float32)),
    )(h, xp, ah0, ah1, ax0, ax1, wn1a, wn1b, bn1.reshape(1, d), wn2,
      bn2.reshape(1, d))


def _pad_edges(src, dst, dummy):
    e = src.shape[0]
    e_pad = -(-e // _GRAN) * _GRAN
    pad = e_pad - e
    if pad:
        src = jnp.concatenate([src, jnp.zeros((pad,), jnp.int32)])
        dst = jnp.concatenate([dst, jnp.full((pad,), dummy, jnp.int32)])
    return src, dst


def kernel(h_lig, h_rec, x_lig, x_rec, edge_index_ll, edge_index_rl,
           W1e_ll, b1e_ll, W2e_ll, b2e_ll, W1c_ll, b1c_ll, W2c_ll, b2c_ll,
           W1e_rl, b1e_rl, W2e_rl, b2e_rl, W1c_rl, b1c_rl, W2c_rl, b2c_rl,
           Wn1, bn1, Wn2, bn2):
    n_lig, d = h_lig.shape

    w_lig = jnp.concatenate(
        [W1e_ll[:d], W1c_ll[:d], W1e_ll[d:2 * d], W1c_ll[d:2 * d],
         W1e_rl[d:2 * d], W1c_rl[d:2 * d]], axis=1)
    b_lig = jnp.concatenate(
        [jnp.zeros((2 * d,), jnp.float32), b1e_ll, b1c_ll, b1e_rl, b1c_rl])
    x_lig_p = jnp.pad(x_lig, ((0, 0), (0, d - x_lig.shape[1])))
    x_rec_p = jnp.pad(x_rec, ((0, 0), (0, d - x_rec.shape[1])))
    t_src_ll, t_dst_ll, t_dst_rl = _rowmm(h_lig, w_lig, b_lig, 3)
    w_rec = jnp.concatenate([W1e_rl[:d], W1c_rl[:d]], axis=1)
    (t_src_rl,) = _rowmm(h_rec, w_rec, jnp.zeros((2 * d,), jnp.float32), 1)

    src_ll, dst_ll = _pad_edges(edge_index_ll[0], edge_index_ll[1], n_lig)
    src_rl, dst_rl = _pad_edges(edge_index_rl[0], edge_index_rl[1], n_lig)
    (gs_ll, gd_ll, xf_ll,
     gs_rl, gd_rl, xf_rl) = _sc_gather(
        t_src_ll, t_dst_ll, x_lig_p, x_lig_p, src_ll, dst_ll,
        t_src_rl, t_dst_rl, x_rec_p, x_lig_p, src_rl, dst_rl)

    mh_ll, mx_ll = _tc_edge(gs_ll, gd_ll, xf_ll,
                            W2e_ll, b2e_ll, W2c_ll[:, 0], b2c_ll,
                            W1e_ll[2 * d], W1c_ll[2 * d])
    mh_rl, mx_rl = _tc_edge(gs_rl, gd_rl, xf_rl,
                            W2e_rl, b2e_rl, W2c_rl[:, 0], b2c_rl,
                            W1e_rl[2 * d], W1c_rl[2 * d])

    nacc = -(-(n_lig + 1) // 128) * 128
    zh = jnp.zeros((nacc, d), jnp.float32)
    acc_h = _sc_scatter(dst_ll, mh_ll, dst_rl, mh_rl, zh)
    acc_x = _sc_scatter(dst_ll, mx_ll, dst_rl, mx_rl, zh)

    h_out, xp_out = _tc_final(
        h_lig, x_lig_p,
        acc_h[0, :n_lig], acc_h[1, :n_lig], acc_x[0, :n_lig], acc_x[1, :n_lig],
        Wn1[:d], Wn1[d:], bn1, Wn2, bn2)

    return (h_out, h_rec, xp_out[:, :x_lig.shape[1]], x_rec)
